# Initial kernel scaffold; baseline (speedup 1.0000x reference)
import jax
import jax.numpy as jnp
from jax.experimental import pallas as pl


def kernel(x, target):
    raise NotImplementedError("write your pallas kernel here")


# speedup vs baseline: 6.5547x; 6.5547x over previous
import functools
import math

import jax
import jax.numpy as jnp
from jax.experimental import pallas as pl
from jax.experimental.pallas import tpu as pltpu

_N = 2048
_V = 32000
_PAD = 0
_SMOOTH = 0.1
_CONF = 1.0 - _SMOOTH
_S = _SMOOTH / (_V - 2)
_K = (_V - 2) * _S * math.log(_S) + _CONF * math.log(_CONF)

_R = 256
_C = 6400


def _loss_body(t_ref, x_ref, o_ref, *, nr, nc):
    i = pl.program_id(0)
    j = pl.program_id(1)

    @pl.when((i == 0) & (j == 0))
    def _init():
        o_ref[0, 0] = 0.0

    t = t_ref[...]
    valid = (t != _PAD)
    xb = x_ref[...]

    cols = j * _C + jax.lax.broadcasted_iota(jnp.int32, (_R, _C), 1)
    match = cols == t
    w = (jnp.float32(_S - _CONF) * match.astype(jnp.float32)
         + jnp.float32(_S) * (cols == 0).astype(jnp.float32)
         - jnp.float32(_S))
    contrib = jnp.where(valid, xb * w, 0.0)
    partial = jnp.sum(contrib)

    @pl.when(j == 0)
    def _const():
        o_ref[0, 0] += jnp.float32(_K) * jnp.sum(valid.astype(jnp.float32))

    o_ref[0, 0] += partial


def kernel(x, target):
    nr = _N // _R
    nc = _V // _C
    t2 = target.astype(jnp.int32).reshape(_N, 1)
    out = pl.pallas_call(
        functools.partial(_loss_body, nr=nr, nc=nc),
        grid=(nr, nc),
        in_specs=[
            pl.BlockSpec((_R, 1), lambda i, j: (i, 0)),
            pl.BlockSpec((_R, _C), lambda i, j: (i, j)),
        ],
        out_specs=pl.BlockSpec((1, 1), lambda i, j: (0, 0),
                               memory_space=pltpu.SMEM),
        out_shape=jax.ShapeDtypeStruct((1, 1), jnp.float32),
        compiler_params=pltpu.CompilerParams(
            dimension_semantics=("arbitrary", "arbitrary")),
    )(t2, x)
    return out[0, 0]


# speedup vs baseline: 6.5766x; 1.0033x over previous
import functools
import math

import jax
import jax.numpy as jnp
from jax.experimental import pallas as pl
from jax.experimental.pallas import tpu as pltpu

_N = 2048
_V = 32000
_PAD = 0
_SMOOTH = 0.1
_CONF = 1.0 - _SMOOTH
_S = _SMOOTH / (_V - 2)
_K = (_V - 2) * _S * math.log(_S) + _CONF * math.log(_CONF)

_R = 256
_C = 6400


def _loss_body(t_ref, x_ref, o_ref, *, nr, nc):
    i = pl.program_id(0)
    j = pl.program_id(1)

    @pl.when((i == 0) & (j == 0))
    def _init():
        o_ref[0, 0] = 0.0

    t = t_ref[...]
    valid = (t != _PAD)
    xb = x_ref[...]

    tloc = jnp.where(valid, t - j * _C, -1)
    iota = jax.lax.broadcasted_iota(jnp.int32, (_R, _C), 1)
    gath = jnp.sum(jnp.where(iota == tloc, xb, 0.0))

    rows = jnp.sum(xb, axis=1, keepdims=True)
    rsum = jnp.sum(jnp.where(valid, rows, 0.0))

    partial = jnp.float32(_S - _CONF) * gath - jnp.float32(_S) * rsum

    @pl.when(j == 0)
    def _const():
        x0 = xb[:, 0:1]
        o_ref[0, 0] += jnp.sum(
            jnp.where(valid, jnp.float32(_K) + jnp.float32(_S) * x0, 0.0))

    o_ref[0, 0] += partial


def kernel(x, target):
    nr = _N // _R
    nc = _V // _C
    t2 = target.astype(jnp.int32).reshape(_N, 1)
    out = pl.pallas_call(
        functools.partial(_loss_body, nr=nr, nc=nc),
        grid=(nr, nc),
        in_specs=[
            pl.BlockSpec((_R, 1), lambda i, j: (i, 0)),
            pl.BlockSpec((_R, _C), lambda i, j: (i, j)),
        ],
        out_specs=pl.BlockSpec((1, 1), lambda i, j: (0, 0),
                               memory_space=pltpu.SMEM),
        out_shape=jax.ShapeDtypeStruct((1, 1), jnp.float32),
        compiler_params=pltpu.CompilerParams(
            dimension_semantics=("arbitrary", "arbitrary")),
    )(t2, x)
    return out[0, 0]
